# baseline (speedup 1.0000x reference)
import jax
import jax.numpy as jnp
from jax.experimental import pallas as pl


def _seg_softmax(logits, seg, n):
    mx = jax.ops.segment_max(logits, seg, num_segments=n)
    ex = jnp.exp(logits - mx[seg])
    den = jax.ops.segment_sum(ex, seg, num_segments=n)
    return ex / (den[seg] + 1e-16)


def _gat(x, src, dst, efeat, Wl, Wr, We, a, b):
    n = x.shape[0]
    H, dh = a.shape
    xl = x @ Wl
    xr = x @ Wr
    ee = efeat @ We
    m = jax.nn.leaky_relu(xl[src] + xr[dst] + ee, 0.2).reshape(-1, H, dh)
    logit = jnp.einsum('ehd,hd->eh', m, a)
    alpha = _seg_softmax(logit, dst, n)
    msg = xl[src].reshape(-1, H, dh) * alpha[:, :, None]
    return jax.ops.segment_sum(msg.reshape(-1, H * dh), dst, num_segments=n) + b


def _lstm(seq, Wih, Whh, b):
    B = seq.shape[0]
    Hd = Whh.shape[1]

    def step(carry, xt):
        h, c = carry
        z = xt @ Wih.T + h @ Whh.T + b
        i, f, g, o = jnp.split(z, 4, axis=-1)
        c = jax.nn.sigmoid(f) * c + jax.nn.sigmoid(i) * jnp.tanh(g)
        h = jax.nn.sigmoid(o) * jnp.tanh(c)
        return (h, c), h

    init = (jnp.zeros((B, Hd), seq.dtype), jnp.zeros((B, Hd), seq.dtype))
    _, hs = jax.lax.scan(step, init, jnp.swapaxes(seq, 0, 1))
    return jnp.swapaxes(hs, 0, 1)


def kernel(x, edge_index, edge_attr, M, params):
    p = params
    src = edge_index[0]
    dst = edge_index[1]
    outs = []
    for t in range(x.shape[1]):
        h = _gat(x[:, t, :], src, dst, edge_attr, p['Wl1'], p['Wr1'], p['We1'], p['a1'], p['b1'])
        h = jax.nn.elu(h)
        h = _gat(h, src, dst, edge_attr, p['Wl2'], p['Wr2'], p['We2'], p['a2'], p['b2'])
        h = jax.nn.elu(h)
        outs.append(h)
    Q = jnp.stack(outs, axis=1)
    G = jnp.einsum('ng,ntd->gtd', M, Q)
    hf = _lstm(G, p['Wih_f'], p['Whh_f'], p['bf'])
    hb = _lstm(G[:, ::-1, :], p['Wih_b'], p['Whh_b'], p['bb'])[:, ::-1, :]
    hcat = jnp.concatenate([hf, hb], axis=-1)
    return (hcat @ p['Wo'] + p['bo'])[..., 0]
